# baseline (speedup 1.0000x reference)
import functools
import math

import jax
import jax.numpy as jnp
from jax import lax
from jax.experimental import pallas as pl
from jax.experimental.pallas import tpu as pltpu
from jax.experimental.pallas import tpu_sc as plsc

_CHUNK = 128
_K = 4
_NSLOT = 3


def kernel(x, W):
    B, F = x.shape
    V, D = W.shape
    N = B * F
    scale = math.sqrt(D)

    info = plsc.get_sparse_core_info()
    NC, NS = info.num_cores, info.num_subcores
    NW = NC * NS
    b_per_w = N // NW
    rows_per_step = _K * _CHUNK
    n_steps = b_per_w // rows_per_step

    idx = x.reshape(N).astype(jnp.int32)

    mesh = plsc.VectorSubcoreMesh(core_axis_name="c", subcore_axis_name="s")

    @functools.partial(
        pl.kernel,
        out_type=jax.ShapeDtypeStruct((N, D), W.dtype),
        mesh=mesh,
        compiler_params=pltpu.CompilerParams(use_tc_tiling_on_sc=False),
        scratch_types=[
            pltpu.VMEM((b_per_w,), jnp.int32),
            [pltpu.VMEM((rows_per_step, D), jnp.float32)] * _NSLOT,
            [pltpu.SemaphoreType.DMA] * _NSLOT,
            [pltpu.SemaphoreType.DMA] * _NSLOT,
        ],
    )
    def sc_embed(idx_hbm, w_hbm, out_hbm, idx_v, bufs, gsems, wsems):
        wid = lax.axis_index("s") * NC + lax.axis_index("c")
        base = wid * b_per_w
        pltpu.sync_copy(idx_hbm.at[pl.ds(base, b_per_w)], idx_v)

        def gather(step, s, j):
            off = pl.multiple_of(step * rows_per_step + j * _CHUNK, _CHUNK)
            return pltpu.make_async_copy(
                w_hbm.at[idx_v.at[pl.ds(off, _CHUNK)]],
                bufs[s].at[pl.ds(j * _CHUNK, _CHUNK)],
                gsems[s],
            )

        def writeback(step, s):
            off = pl.multiple_of(step * rows_per_step, rows_per_step)
            return pltpu.make_async_copy(
                bufs[s], out_hbm.at[pl.ds(base + off, rows_per_step)], wsems[s]
            )

        for j in range(_K):
            gather(0, 0, j).start()
        for j in range(_K):
            gather(1, 1, j).start()

        @pl.loop(0, n_steps)
        def _(h):
            for s in range(_NSLOT):

                @pl.when(h % _NSLOT == s)
                def _():
                    for j in range(_K):
                        gather(h, s, j).wait()

                    writeback(h, s).start()

                    s2 = (s + 2) % _NSLOT

                    @pl.when(h + 2 < n_steps)
                    def _():
                        @pl.when(h >= 1)
                        def _():
                            writeback(h - 1, s2).wait()

                        for j in range(_K):
                            gather(h + 2, s2, j).start()

        for s in range(_NSLOT):
            last = n_steps - 1 - (n_steps - 1 - s) % _NSLOT
            writeback(last, s).wait()

    out = sc_embed(idx, W)
    return (out * scale).reshape(B, F, D)


# speedup vs baseline: 1.3549x; 1.3549x over previous
import functools
import math

import jax
import jax.numpy as jnp
from jax import lax
from jax.experimental import pallas as pl
from jax.experimental.pallas import tpu as pltpu
from jax.experimental.pallas import tpu_sc as plsc

_LANES = 16
_CHUNK = 128
_K = 4
_NSLOT = 3


def kernel(x, W):
    B, F = x.shape
    V, D = W.shape
    N = B * F
    scale = math.sqrt(D)

    info = plsc.get_sparse_core_info()
    NC, NS = info.num_cores, info.num_subcores
    NW = NC * NS
    b_per_w = N // NW
    rows_per_step = _K * _CHUNK
    n_steps = b_per_w // rows_per_step

    idx = x.reshape(N).astype(jnp.int32)

    mesh = plsc.VectorSubcoreMesh(core_axis_name="c", subcore_axis_name="s")

    @functools.partial(
        pl.kernel,
        out_type=jax.ShapeDtypeStruct((N, D), W.dtype),
        mesh=mesh,
        compiler_params=pltpu.CompilerParams(use_tc_tiling_on_sc=False),
        scratch_types=[
            pltpu.VMEM((b_per_w,), jnp.int32),
            [pltpu.VMEM((rows_per_step, D), jnp.float32)] * _NSLOT,
            [pltpu.SemaphoreType.DMA] * _NSLOT,
            [pltpu.SemaphoreType.DMA] * _NSLOT,
        ],
    )
    def sc_embed(idx_hbm, w_hbm, out_hbm, idx_v, bufs, gsems, wsems):
        wid = lax.axis_index("s") * NC + lax.axis_index("c")
        base = wid * b_per_w
        pltpu.sync_copy(idx_hbm.at[pl.ds(base, b_per_w)], idx_v)

        def gather(step, s, j):
            off = pl.multiple_of(step * rows_per_step + j * _CHUNK, _CHUNK)
            return pltpu.make_async_copy(
                w_hbm.at[idx_v.at[pl.ds(off, _CHUNK)]],
                bufs[s].at[pl.ds(j * _CHUNK, _CHUNK)],
                gsems[s],
            )

        def writeback(step, s):
            off = pl.multiple_of(step * rows_per_step, rows_per_step)
            return pltpu.make_async_copy(
                bufs[s], out_hbm.at[pl.ds(base + off, rows_per_step)], wsems[s]
            )

        for j in range(_K):
            gather(0, 0, j).start()
        for j in range(_K):
            gather(1, 1, j).start()

        @pl.loop(0, n_steps)
        def _(h):
            for s in range(_NSLOT):

                @pl.when(h % _NSLOT == s)
                def _():
                    for j in range(_K):
                        gather(h, s, j).wait()

                    @pl.loop(0, rows_per_step)
                    def _(r):
                        for c in range(D // _LANES):
                            sl = (r, pl.ds(c * _LANES, _LANES))
                            bufs[s][sl] = bufs[s][sl] * scale

                    writeback(h, s).start()

                    s2 = (s + 2) % _NSLOT

                    @pl.when(h + 2 < n_steps)
                    def _():
                        @pl.when(h >= 1)
                        def _():
                            writeback(h - 1, s2).wait()

                        for j in range(_K):
                            gather(h + 2, s2, j).start()

        for s in range(_NSLOT):
            last = n_steps - 1 - (n_steps - 1 - s) % _NSLOT
            writeback(last, s).wait()

    out = sc_embed(idx, W)
    return out.reshape(B, F, D)


# speedup vs baseline: 1.3613x; 1.0047x over previous
import functools
import math

import jax
import jax.numpy as jnp
from jax import lax
from jax.experimental import pallas as pl
from jax.experimental.pallas import tpu as pltpu
from jax.experimental.pallas import tpu_sc as plsc

_LANES = 16
_CHUNK = 128
_K = 2
_NSLOT = 6


def kernel(x, W):
    B, F = x.shape
    V, D = W.shape
    N = B * F
    scale = math.sqrt(D)

    info = plsc.get_sparse_core_info()
    NC, NS = info.num_cores, info.num_subcores
    NW = NC * NS
    b_per_w = N // NW
    rows_per_step = _K * _CHUNK
    n_steps = b_per_w // rows_per_step

    idx = x.reshape(N).astype(jnp.int32)

    mesh = plsc.VectorSubcoreMesh(core_axis_name="c", subcore_axis_name="s")

    @functools.partial(
        pl.kernel,
        out_type=jax.ShapeDtypeStruct((N, D), W.dtype),
        mesh=mesh,
        compiler_params=pltpu.CompilerParams(use_tc_tiling_on_sc=False),
        scratch_types=[
            pltpu.VMEM((b_per_w,), jnp.int32),
            [pltpu.VMEM((rows_per_step, D), jnp.float32)] * _NSLOT,
            [pltpu.SemaphoreType.DMA] * _NSLOT,
            [pltpu.SemaphoreType.DMA] * _NSLOT,
        ],
    )
    def sc_embed(idx_hbm, w_hbm, out_hbm, idx_v, bufs, gsems, wsems):
        wid = lax.axis_index("s") * NC + lax.axis_index("c")
        base = wid * b_per_w
        pltpu.sync_copy(idx_hbm.at[pl.ds(base, b_per_w)], idx_v)

        def gather(step, s, j):
            off = pl.multiple_of(step * rows_per_step + j * _CHUNK, _CHUNK)
            return pltpu.make_async_copy(
                w_hbm.at[idx_v.at[pl.ds(off, _CHUNK)]],
                bufs[s].at[pl.ds(j * _CHUNK, _CHUNK)],
                gsems[s],
            )

        def writeback(step, s):
            off = pl.multiple_of(step * rows_per_step, rows_per_step)
            return pltpu.make_async_copy(
                bufs[s], out_hbm.at[pl.ds(base + off, rows_per_step)], wsems[s]
            )

        for t in range(_NSLOT - 1):
            for j in range(_K):
                gather(t, t, j).start()

        @pl.loop(0, n_steps)
        def _(h):
            for s in range(_NSLOT):

                @pl.when(h % _NSLOT == s)
                def _():
                    for j in range(_K):
                        gather(h, s, j).wait()

                    @pl.loop(0, rows_per_step)
                    def _(r):
                        for c in range(D // _LANES):
                            sl = (r, pl.ds(c * _LANES, _LANES))
                            bufs[s][sl] = bufs[s][sl] * scale

                    writeback(h, s).start()

                    s2 = (s + _NSLOT - 1) % _NSLOT

                    @pl.when(h + _NSLOT - 1 < n_steps)
                    def _():
                        @pl.when(h >= 1)
                        def _():
                            writeback(h - 1, s2).wait()

                        for j in range(_K):
                            gather(h + _NSLOT - 1, s2, j).start()

        for s in range(_NSLOT):
            last = n_steps - 1 - (n_steps - 1 - s) % _NSLOT
            writeback(last, s).wait()

    out = sc_embed(idx, W)
    return out.reshape(B, F, D)
